# Initial kernel scaffold; baseline (speedup 1.0000x reference)
import functools

import jax
import jax.numpy as jnp
from jax import lax
from jax.experimental import pallas as pl
from jax.experimental.pallas import tpu as pltpu
from jax.experimental.pallas import tpu_sc as plsc

N_NODES = 100000
HIDDEN = 128
N_PAD = 102400
PER_W = 3200
CHUNK = 128
N_CHUNKS = PER_W // CHUNK


def _body(in_idx, out_idx, w_in, w_out, out, idx_a, idx_b, buf_a, buf_b, sem_a,
          sem_b):
  nc = 2
  wid = lax.axis_index("s") * nc + lax.axis_index("c")
  row0 = wid * N_CHUNKS

  pltpu.sync_copy(in_idx.at[pl.ds(row0, N_CHUNKS)], idx_a)
  pltpu.sync_copy(out_idx.at[pl.ds(row0, N_CHUNKS)], idx_b)

  def chunk_body(j, carry):
    ca = pltpu.async_copy(w_in.at[idx_a.at[j]], buf_a, sem_a)
    cb = pltpu.async_copy(w_out.at[idx_b.at[j]], buf_b, sem_b)
    ca.wait()
    cb.wait()

    def row_body(r, c):
      for k in range(HIDDEN // 16):
        s = pl.ds(k * 16, 16)
        buf_a[r, s] = buf_a[r, s] + buf_b[r, s]
      return c

    lax.fori_loop(0, CHUNK, row_body, 0)

    base = wid * PER_W + j * CHUNK
    pltpu.sync_copy(buf_a, out.at[pl.ds(base, CHUNK)])
    return carry

  lax.fori_loop(0, N_CHUNKS, chunk_body, 0)


@jax.jit
def kernel(in_deg, out_deg, W_in, W_out):
  pad = N_PAD - N_NODES
  in_p = jnp.pad(in_deg.astype(jnp.int32), (0, pad)).reshape(-1, CHUNK)
  out_p = jnp.pad(out_deg.astype(jnp.int32), (0, pad)).reshape(-1, CHUNK)

  mesh = plsc.VectorSubcoreMesh(core_axis_name="c", subcore_axis_name="s")
  f = pl.kernel(
      _body,
      out_type=jax.ShapeDtypeStruct((N_PAD, HIDDEN), jnp.float32),
      mesh=mesh,
      scratch_types=[
          pltpu.VMEM((N_CHUNKS, CHUNK), jnp.int32),
          pltpu.VMEM((N_CHUNKS, CHUNK), jnp.int32),
          pltpu.VMEM((CHUNK, HIDDEN), jnp.float32),
          pltpu.VMEM((CHUNK, HIDDEN), jnp.float32),
          pltpu.SemaphoreType.DMA,
          pltpu.SemaphoreType.DMA,
      ],
  )
  res = f(in_p, out_p, W_in, W_out)
  return res[:N_NODES]


# speedup vs baseline: 1.4803x; 1.4803x over previous
import functools

import jax
import jax.numpy as jnp
from jax import lax
from jax.experimental import pallas as pl
from jax.experimental.pallas import tpu as pltpu
from jax.experimental.pallas import tpu_sc as plsc

N_NODES = 100000
HIDDEN = 128
N_PAD = 102400
PER_W = 3200
CHUNK = 128
N_CHUNKS = PER_W // CHUNK


def _body(in_idx, out_idx, w_in, w_out, out, idx_a, idx_b, buf_a, buf_b, sem_a,
          sem_b):
  nc = 2
  wid = lax.axis_index("s") * nc + lax.axis_index("c")

  pltpu.sync_copy(in_idx.at[pl.ds(wid * PER_W, PER_W)], idx_a)
  pltpu.sync_copy(out_idx.at[pl.ds(wid * PER_W, PER_W)], idx_b)

  def chunk_body(j, carry):
    ia = idx_a.at[pl.ds(j * CHUNK, CHUNK)]
    ib = idx_b.at[pl.ds(j * CHUNK, CHUNK)]
    ca = pltpu.async_copy(w_in.at[ia], buf_a, sem_a)
    cb = pltpu.async_copy(w_out.at[ib], buf_b, sem_b)
    ca.wait()
    cb.wait()

    def row_body(r, c):
      for k in range(HIDDEN // 16):
        s = pl.ds(k * 16, 16)
        buf_a[r, s] = buf_a[r, s] + buf_b[r, s]
      return c

    lax.fori_loop(0, CHUNK, row_body, 0)

    base = wid * PER_W + j * CHUNK
    pltpu.sync_copy(buf_a, out.at[pl.ds(base, CHUNK)])
    return carry

  lax.fori_loop(0, N_CHUNKS, chunk_body, 0)


@jax.jit
def kernel(in_deg, out_deg, W_in, W_out):
  pad = N_PAD - N_NODES
  in_p = jnp.pad(in_deg.astype(jnp.int32), (0, pad))
  out_p = jnp.pad(out_deg.astype(jnp.int32), (0, pad))

  mesh = plsc.VectorSubcoreMesh(core_axis_name="c", subcore_axis_name="s")
  f = pl.kernel(
      _body,
      out_type=jax.ShapeDtypeStruct((N_PAD, HIDDEN), jnp.float32),
      mesh=mesh,
      scratch_types=[
          pltpu.VMEM((PER_W,), jnp.int32),
          pltpu.VMEM((PER_W,), jnp.int32),
          pltpu.VMEM((CHUNK, HIDDEN), jnp.float32),
          pltpu.VMEM((CHUNK, HIDDEN), jnp.float32),
          pltpu.SemaphoreType.DMA,
          pltpu.SemaphoreType.DMA,
      ],
  )
  res = f(in_p, out_p, W_in, W_out)
  return res[:N_NODES]


# speedup vs baseline: 1.6625x; 1.1231x over previous
import jax
import jax.numpy as jnp
from jax import lax
from jax.experimental import pallas as pl
from jax.experimental.pallas import tpu as pltpu
from jax.experimental.pallas import tpu_sc as plsc

N_NODES = 100000
HIDDEN = 128
N_PAD = 102400
PER_W = 3200
CHUNK = 128
N_CHUNKS = PER_W // CHUNK


def _body(in_idx, out_idx, w_in, w_out, out, idx_a, idx_b, ba0, ba1, bb0, bb1,
          sa0, sa1, sb0, sb1):
  nc = 2
  wid = lax.axis_index("s") * nc + lax.axis_index("c")

  pltpu.sync_copy(in_idx.at[pl.ds(wid * PER_W, PER_W)], idx_a)
  pltpu.sync_copy(out_idx.at[pl.ds(wid * PER_W, PER_W)], idx_b)

  bufs = ((ba0, bb0, sa0, sb0), (ba1, bb1, sa1, sb1))

  def issue(j, slot):
    ba, bb, sa, sb = bufs[slot]
    ia = idx_a.at[pl.ds(j * CHUNK, CHUNK)]
    ib = idx_b.at[pl.ds(j * CHUNK, CHUNK)]
    pltpu.async_copy(w_in.at[ia], ba, sa)
    pltpu.async_copy(w_out.at[ib], bb, sb)

  def finish(j, slot):
    ba, bb, sa, sb = bufs[slot]
    ia = idx_a.at[pl.ds(j * CHUNK, CHUNK)]
    ib = idx_b.at[pl.ds(j * CHUNK, CHUNK)]
    pltpu.make_async_copy(w_in.at[ia], ba, sa).wait()
    pltpu.make_async_copy(w_out.at[ib], bb, sb).wait()

    @plsc.parallel_loop(0, CHUNK, unroll=4)
    def _(r):
      for k in range(HIDDEN // 16):
        s = pl.ds(k * 16, 16)
        plsc.addupdate(ba.at[r, s], bb[r, s])

    pltpu.sync_copy(ba, out.at[pl.ds(wid * PER_W + j * CHUNK, CHUNK)])

  issue(0, 0)

  def pair_body(p, carry):
    for s in range(2):
      j = 2 * p + s

      @pl.when(j + 1 < N_CHUNKS)
      def _():
        issue(j + 1, 1 - s)

      finish(j, s)
    return carry

  lax.fori_loop(0, N_CHUNKS // 2, pair_body, 0)
  finish(N_CHUNKS - 1, (N_CHUNKS - 1) % 2)


@jax.jit
def kernel(in_deg, out_deg, W_in, W_out):
  pad = N_PAD - N_NODES
  in_p = jnp.pad(in_deg.astype(jnp.int32), (0, pad))
  out_p = jnp.pad(out_deg.astype(jnp.int32), (0, pad))

  mesh = plsc.VectorSubcoreMesh(core_axis_name="c", subcore_axis_name="s")
  f = pl.kernel(
      _body,
      out_type=jax.ShapeDtypeStruct((N_PAD, HIDDEN), jnp.float32),
      mesh=mesh,
      scratch_types=[
          pltpu.VMEM((PER_W,), jnp.int32),
          pltpu.VMEM((PER_W,), jnp.int32),
          pltpu.VMEM((CHUNK, HIDDEN), jnp.float32),
          pltpu.VMEM((CHUNK, HIDDEN), jnp.float32),
          pltpu.VMEM((CHUNK, HIDDEN), jnp.float32),
          pltpu.VMEM((CHUNK, HIDDEN), jnp.float32),
          pltpu.SemaphoreType.DMA,
          pltpu.SemaphoreType.DMA,
          pltpu.SemaphoreType.DMA,
          pltpu.SemaphoreType.DMA,
      ],
  )
  res = f(in_p, out_p, W_in, W_out)
  return res[:N_NODES]
